# baseline (speedup 1.0000x reference)
import functools

import jax
import jax.numpy as jnp
from jax.experimental import pallas as pl
from jax.experimental.pallas import tpu as pltpu

N = 10000
E = 160000
D = 256


def _mm_body(h_ref, w_ref, a2_ref, z_ref, st_ref):
    z = jnp.dot(h_ref[...], w_ref[...], preferred_element_type=jnp.float32)
    z_ref[...] = z
    st_ref[...] = jnp.dot(z, a2_ref[...], preferred_element_type=jnp.float32)


def _project(h, W, A2):
    blk = 1000
    return pl.pallas_call(
        _mm_body,
        grid=(N // blk,),
        in_specs=[
            pl.BlockSpec((blk, D), lambda i: (i, 0)),
            pl.BlockSpec((D, D), lambda i: (0, 0)),
            pl.BlockSpec((D, 128), lambda i: (0, 0)),
        ],
        out_specs=[
            pl.BlockSpec((blk, D), lambda i: (i, 0)),
            pl.BlockSpec((blk, 128), lambda i: (i, 0)),
        ],
        out_shape=[
            jax.ShapeDtypeStruct((N, D), jnp.float32),
            jax.ShapeDtypeStruct((N, 128), jnp.float32),
        ],
    )(h, W, A2)


def kernel(h, edge_index, W, a):
    A2 = jnp.zeros((D, 128), jnp.float32)
    A2 = A2.at[:, 0].set(a[:D]).at[:, 1].set(a[D:])
    z, st = _project(h, W, A2)
    s = st[:, 0]
    t = st[:, 1]
    src = edge_index[0]
    dst = edge_index[1]

    e = jax.nn.leaky_relu(s[src] + t[dst], negative_slope=0.01)
    ones = jnp.ones_like(e)
    deg = jax.ops.segment_sum(ones, dst, num_segments=N)
    sum_e = jax.ops.segment_sum(e, dst, num_segments=N)
    mean_e = sum_e / jnp.maximum(deg, 1.0)
    e_masked = jnp.where(e < mean_e[dst], jnp.float32(0.0), e)
    M = jnp.maximum(jnp.max(e), 0.0)
    ex = jnp.exp(e_masked - M)
    denom = jax.ops.segment_sum(ex, dst, num_segments=N)
    alpha = ex / denom[dst]
    h_out = jax.ops.segment_sum(alpha[:, None] * z[src], dst, num_segments=N)
    return h_out


# speedup vs baseline: 2.9972x; 2.9972x over previous
import functools

import jax
import jax.numpy as jnp
from jax import lax
from jax.experimental import pallas as pl
from jax.experimental.pallas import tpu as pltpu
from jax.experimental.pallas import tpu_sc as plsc

N = 10000
E = 160000
D = 256

NP_ROWS = 96
EC = E // 16
CH = 1280
NCH = E // CH
SB = 64
RT = 312

_CP = pltpu.CompilerParams(needs_layout_passes=False)

def _z16f():
    return jnp.zeros((16,), jnp.float32)


def _mm_body(h_ref, w_ref, a2_ref, z_ref, st_ref):
    z = jnp.dot(h_ref[...], w_ref[...], preferred_element_type=jnp.float32)
    z_ref[...] = z
    st_ref[...] = jnp.dot(z, a2_ref[...], preferred_element_type=jnp.float32)


def _project(h, W, A2):
    blk = 1000
    return pl.pallas_call(
        _mm_body,
        grid=(N // blk,),
        in_specs=[
            pl.BlockSpec((blk, D), lambda i: (i, 0)),
            pl.BlockSpec((D, D), lambda i: (0, 0)),
            pl.BlockSpec((D, 128), lambda i: (0, 0)),
        ],
        out_specs=[
            pl.BlockSpec((blk, D), lambda i: (i, 0)),
            pl.BlockSpec((blk, 128), lambda i: (i, 0)),
        ],
        out_shape=[
            jax.ShapeDtypeStruct((N, D), jnp.float32),
            jax.ShapeDtypeStruct((N, 128), jnp.float32),
        ],
    )(h, W, A2)


def _zero2d(ref, nrows):
    ncol16 = ref.shape[1] // 16

    def row(r, carry):
        for q in range(ncol16):
            ref[r, pl.ds(q * 16, 16)] = _z16f()
        return carry

    lax.fori_loop(0, nrows, row, 0)


def _sc_body(z_hbm, s_hbm, t_hbm, src_hbm, dst_hbm, out_hbm,
             sh_deg, sh_sum, sh_den, sh_max, sh_ex, sem_a, sem_b, sem_g):
    c = lax.axis_index("c")
    sidx = lax.axis_index("s")
    e0 = sidx * EC

    def scalar_phase(sv, tv, srcv, dstv, ev, la, lb, idx80, mxb):
        pltpu.sync_copy(s_hbm, sv)
        pltpu.sync_copy(t_hbm, tv)
        pltpu.sync_copy(src_hbm.at[pl.ds(e0, EC)], srcv)
        pltpu.sync_copy(dst_hbm.at[pl.ds(e0, EC)], dstv)
        _zero2d(la, NP_ROWS)
        _zero2d(lb, NP_ROWS)

        @pl.when(sidx == 0)
        def _():
            pltpu.sync_copy(la, sh_deg)
            pltpu.sync_copy(la, sh_sum)
            pltpu.sync_copy(la, sh_den)

        def fill_idx(j, carry):
            idx80[pl.ds(j * 16, 16)] = lax.iota(jnp.int32, 16) + j * 16
            return carry

        lax.fori_loop(0, NP_ROWS // 16, fill_idx, 0)
        plsc.subcore_barrier()

        def edge_step(i, mx):
            sv16 = srcv[pl.ds(i * 16, 16)]
            dv16 = dstv[pl.ds(i * 16, 16)]
            ss = plsc.load_gather(sv, [sv16])
            tt = plsc.load_gather(tv, [dv16])
            x = ss + tt
            e = jnp.where(x >= 0.0, x, 0.01 * x)
            ev[pl.ds(i * 16, 16)] = e
            dr = lax.shift_right_logical(dv16, 7)
            dc = lax.bitwise_and(dv16, 127)
            plsc.addupdate_scatter(la, [dr, dc], jnp.ones((16,), jnp.float32))
            plsc.addupdate_scatter(lb, [dr, dc], e)
            return jnp.maximum(mx, e)

        mxv = lax.fori_loop(0, EC // 16, edge_step,
                            jnp.full((16,), -1e30, jnp.float32))
        mxb[...] = mxv
        pltpu.sync_copy(mxb, sh_max.at[sidx])
        pltpu.sync_copy(la, sh_deg.at[idx80], add=True)
        pltpu.sync_copy(lb, sh_sum.at[idx80], add=True)
        plsc.subcore_barrier()

        pltpu.sync_copy(sh_deg, la)
        pltpu.sync_copy(sh_sum, lb)

        def mean_row(r, carry):
            for q in range(8):
                dg = la[r, pl.ds(q * 16, 16)]
                sm = lb[r, pl.ds(q * 16, 16)]
                la[r, pl.ds(q * 16, 16)] = sm / jnp.maximum(dg, 1.0)
            return carry

        lax.fori_loop(0, NP_ROWS, mean_row, 0)

        _zero2d(lb, NP_ROWS)

        def mask_step(i, mm):
            dv16 = dstv[pl.ds(i * 16, 16)]
            e = ev[pl.ds(i * 16, 16)]
            dr = lax.shift_right_logical(dv16, 7)
            dc = lax.bitwise_and(dv16, 127)
            mn = plsc.load_gather(la, [dr, dc])
            em = jnp.where(e < mn, 0.0, e)
            ex = jnp.exp(em - mm)
            ev[pl.ds(i * 16, 16)] = ex
            plsc.addupdate_scatter(lb, [dr, dc], ex)
            return mm

        return mask_step

    def scope1(sv, tv, srcv, dstv, ev, la, lb, idx80, mxb, mx2):
        mask_step = scalar_phase(sv, tv, srcv, dstv, ev, la, lb, idx80, mxb)
        pltpu.sync_copy(sh_max, mx2)
        mm = jnp.full((16,), 0.0, jnp.float32)
        for r in range(16):
            mm = jnp.maximum(mm, mx2[r, :])
        m_scalar = jnp.max(mm)
        mmv = jnp.full((16,), 1.0, jnp.float32) * m_scalar
        lax.fori_loop(0, EC // 16, mask_step, mmv)
        pltpu.sync_copy(lb, sh_den.at[idx80], add=True)
        pltpu.sync_copy(ev, sh_ex.at[pl.ds(e0, EC)])
        plsc.subcore_barrier()

    pl.run_scoped(
        scope1,
        pltpu.VMEM((N,), jnp.float32),
        pltpu.VMEM((N,), jnp.float32),
        pltpu.VMEM((EC,), jnp.int32),
        pltpu.VMEM((EC,), jnp.int32),
        pltpu.VMEM((EC,), jnp.float32),
        pltpu.VMEM((NP_ROWS, 128), jnp.float32),
        pltpu.VMEM((NP_ROWS, 128), jnp.float32),
        pltpu.VMEM((NP_ROWS,), jnp.int32),
        pltpu.VMEM((16,), jnp.float32),
        pltpu.VMEM((16, 16), jnp.float32),
    )

    def scope2(acc, rows, dstc, srcc, exc, seld, selsrc, selex, scl, inv, win8):
        w = c * 16 + sidx
        lo = RT * w
        nloc = jnp.where(w == 31, RT + 16, RT)
        hi = lo + nloc
        r0 = lax.shift_right_logical(lo, 7)
        loff = lax.bitwise_and(lo, 127)

        win8[...] = lax.iota(jnp.int32, 16) + r0
        pltpu.sync_copy(sh_den.at[win8], inv)
        for r in range(16):
            for q in range(8):
                v = inv[r, pl.ds(q * 16, 16)]
                inv[r, pl.ds(q * 16, 16)] = 1.0 / v

        _zero2d(acc, RT + 16)

        def z16(j, carry):
            seld[pl.ds(j * 16, 16)] = lax.iota(jnp.int32, 16) * 0
            selsrc[pl.ds(j * 16, 16)] = lax.iota(jnp.int32, 16) * 0
            return carry

        lax.fori_loop(0, CH // 16, z16, 0)

        def issue(jn, slot):
            eb = jn * CH
            pltpu.async_copy(dst_hbm.at[pl.ds(eb, CH)], dstc.at[slot], sem_a)
            pltpu.async_copy(src_hbm.at[pl.ds(eb, CH)], srcc.at[slot], sem_a)
            pltpu.async_copy(sh_ex.at[pl.ds(eb, CH)], exc.at[slot], sem_b)

        def drain(slot):
            pltpu.make_async_copy(dst_hbm.at[pl.ds(0, CH)], dstc.at[slot], sem_a).wait()
            pltpu.make_async_copy(src_hbm.at[pl.ds(0, CH)], srcc.at[slot], sem_a).wait()
            pltpu.make_async_copy(sh_ex.at[pl.ds(0, CH)], exc.at[slot], sem_b).wait()

        issue(0, 0)

        def chunk_step(j, carry):
            slot = lax.rem(j, 2)

            @pl.when(j < NCH - 1)
            def _():
                issue(j + 1, 1 - slot)

            drain(slot)

            def filt(k, pos):
                dv = dstc[slot, pl.ds(k * 16, 16)]
                m = jnp.logical_and(dv >= lo, dv < hi)
                plsc.store_compressed(seld.at[pl.ds(pos, 16)], dv - lo, mask=m)
                plsc.store_compressed(selsrc.at[pl.ds(pos, 16)],
                                      srcc[slot, pl.ds(k * 16, 16)], mask=m)
                plsc.store_compressed(selex.at[pl.ds(pos, 16)],
                                      exc[slot, pl.ds(k * 16, 16)], mask=m)
                cnt16 = plsc.all_reduce_population_count(m)[0]
                return pos + cnt16

            cnt = lax.fori_loop(0, CH // 16, filt, 0)

            def scale16(q, carry2):
                di = seld[pl.ds(q * 16, 16)] + loff
                iv = plsc.load_gather(
                    inv, [lax.shift_right_logical(di, 7), lax.bitwise_and(di, 127)])
                scl[pl.ds(q * 16, 16)] = selex[pl.ds(q * 16, 16)] * iv
                return carry2

            lax.fori_loop(0, (cnt + 15) // 16, scale16, 0)

            def subbatch(b, carry3):
                pltpu.async_copy(
                    z_hbm.at[selsrc.at[pl.ds(b * SB, SB)]], rows, sem_g).wait()

                def edge(i, carry4):
                    gi = b * SB + i
                    gv = jnp.full((16,), 1, jnp.int32) * gi
                    d = plsc.load_gather(seld, [gv])[0]
                    sc = plsc.load_gather(scl, [gv])
                    for q in range(16):
                        plsc.addupdate(acc.at[d, pl.ds(q * 16, 16)],
                                       sc * rows[i, pl.ds(q * 16, 16)])
                    return carry4

                lax.fori_loop(0, jnp.minimum(SB, cnt - b * SB), edge, 0)
                return carry3

            lax.fori_loop(0, (cnt + SB - 1) // SB, subbatch, 0)
            return carry

        lax.fori_loop(0, NCH, chunk_step, 0)

        pltpu.sync_copy(acc.at[pl.ds(0, RT)], out_hbm.at[pl.ds(lo, RT)])

        @pl.when(w == 31)
        def _():
            pltpu.sync_copy(acc.at[pl.ds(RT, 16)], out_hbm.at[pl.ds(N - 16, 16)])

    pl.run_scoped(
        scope2,
        pltpu.VMEM((RT + 16, D), jnp.float32),
        pltpu.VMEM((SB, D), jnp.float32),
        pltpu.VMEM((2, CH), jnp.int32),
        pltpu.VMEM((2, CH), jnp.int32),
        pltpu.VMEM((2, CH), jnp.float32),
        pltpu.VMEM((CH,), jnp.int32),
        pltpu.VMEM((CH,), jnp.int32),
        pltpu.VMEM((CH,), jnp.float32),
        pltpu.VMEM((CH,), jnp.float32),
        pltpu.VMEM((16, 128), jnp.float32),
        pltpu.VMEM((16,), jnp.int32),
    )


def _sc_call(z, s, t, src, dst):
    mesh = plsc.VectorSubcoreMesh(core_axis_name="c", subcore_axis_name="s")
    fn = functools.partial(
        pl.kernel,
        out_type=jax.ShapeDtypeStruct((N, D), jnp.float32),
        mesh=mesh,
        compiler_params=_CP,
        scratch_types=[
            pltpu.VMEM_SHARED((NP_ROWS, 128), jnp.float32),
            pltpu.VMEM_SHARED((NP_ROWS, 128), jnp.float32),
            pltpu.VMEM_SHARED((NP_ROWS, 128), jnp.float32),
            pltpu.VMEM_SHARED((16, 16), jnp.float32),
            pltpu.VMEM_SHARED((E,), jnp.float32),
            pltpu.SemaphoreType.DMA,
            pltpu.SemaphoreType.DMA,
            pltpu.SemaphoreType.DMA,
        ],
    )(_sc_body)
    return fn(z, s, t, src, dst)


def kernel(h, edge_index, W, a):
    A2 = jnp.zeros((D, 128), jnp.float32)
    A2 = A2.at[:, 0].set(a[:D]).at[:, 1].set(a[D:])
    z, st = _project(h, W, A2)
    s = st[:, 0]
    t = st[:, 1]
    src = edge_index[0]
    dst = edge_index[1]
    return _sc_call(z, s, t, src, dst)
